# baseline (speedup 1.0000x reference)
import functools

import jax
import jax.numpy as jnp
from jax.experimental import pallas as pl
from jax.experimental.pallas import tpu as pltpu

HIDDEN = 4096
EXPERTS = 64
K = 8
BLK = 2048
KSPLIT = 2
KCHUNK = HIDDEN // KSPLIT
CHUNK = 256


def _router_body(x_ref, w_ref, rw_ref, idx_ref, acc_ref):
    kstep = pl.program_id(1)
    w = w_ref[...]
    riota = (
        jnp.int32(EXPERTS - 1)
        - jax.lax.broadcasted_iota(jnp.int32, (CHUNK, EXPERTS), 1)
    ).astype(jnp.float32)

    for c in range(BLK // CHUNK):
        sl = pl.ds(c * CHUNK, CHUNK)
        partial = jnp.dot(x_ref[sl, :], w, preferred_element_type=jnp.float32)

        @pl.when(kstep == 0)
        def _():
            acc_ref[sl, :] = partial

        @pl.when(kstep == KSPLIT - 1)
        def _():
            vals = acc_ref[sl, :] + partial
            top_vals = []
            top_ridx = []
            for _ in range(K):
                m = jnp.max(vals, axis=-1, keepdims=True)
                masked_iota = jnp.where(vals == m, riota, -1.0)
                r = jnp.max(masked_iota, axis=-1, keepdims=True)
                top_vals.append(m)
                top_ridx.append(r)
                vals = jnp.where(riota == r, -jnp.inf, vals)

            tv = jnp.concatenate(top_vals, axis=1)
            tr = jnp.concatenate(top_ridx, axis=1)
            e = jnp.exp(tv - tv[:, :1])
            rw_ref[sl, :] = e / jnp.sum(e, axis=-1, keepdims=True)
            idx_ref[sl, :] = (jnp.float32(EXPERTS - 1) - tr).astype(jnp.int32)


@jax.jit
def kernel(hidden_states, gate_weight):
    b, s, d = hidden_states.shape
    n_tok = b * s
    x2d = hidden_states.reshape(n_tok, d)
    wt = gate_weight.T

    grid = (n_tok // BLK, KSPLIT)
    rw, idx = pl.pallas_call(
        _router_body,
        grid=grid,
        in_specs=[
            pl.BlockSpec((BLK, KCHUNK), lambda i, k: (i, k)),
            pl.BlockSpec((KCHUNK, EXPERTS), lambda i, k: (k, 0)),
        ],
        out_specs=[
            pl.BlockSpec((BLK, K), lambda i, k: (i, 0)),
            pl.BlockSpec((BLK, K), lambda i, k: (i, 0)),
        ],
        out_shape=[
            jax.ShapeDtypeStruct((n_tok, K), jnp.float32),
            jax.ShapeDtypeStruct((n_tok, K), jnp.int32),
        ],
        scratch_shapes=[pltpu.VMEM((BLK, EXPERTS), jnp.float32)],
        compiler_params=pltpu.CompilerParams(
            dimension_semantics=("parallel", "arbitrary"),
        ),
    )(x2d, wt)

    return rw.reshape(b, s, K), idx.reshape(b, s, K)


# speedup vs baseline: 1.9434x; 1.9434x over previous
import functools

import jax
import jax.numpy as jnp
from jax.experimental import pallas as pl
from jax.experimental.pallas import tpu as pltpu

HIDDEN = 4096
EXPERTS = 64
K = 8
BLK = 1024
CHUNK = 256


def _router_body(x_ref, w_ref, rw_ref, idx_ref):
    w = w_ref[...]
    riota = (
        jnp.int32(EXPERTS - 1)
        - jax.lax.broadcasted_iota(jnp.int32, (CHUNK, EXPERTS), 1)
    ).astype(jnp.float32)

    for c in range(BLK // CHUNK):
        sl = pl.ds(c * CHUNK, CHUNK)
        logits = jnp.dot(x_ref[sl, :], w, preferred_element_type=jnp.float32)

        vals = logits
        top_vals = []
        top_ridx = []
        for _ in range(K):
            m = jnp.max(vals, axis=-1, keepdims=True)
            masked_iota = jnp.where(vals == m, riota, -1.0)
            r = jnp.max(masked_iota, axis=-1, keepdims=True)
            top_vals.append(m)
            top_ridx.append(r)
            vals = jnp.where(riota == r, -jnp.inf, vals)

        tv = jnp.concatenate(top_vals, axis=1)
        tr = jnp.concatenate(top_ridx, axis=1)
        e = jnp.exp(tv - tv[:, :1])
        rw_ref[sl, :] = e / jnp.sum(e, axis=-1, keepdims=True)
        idx_ref[sl, :] = (jnp.float32(EXPERTS - 1) - tr).astype(jnp.int32)


@jax.jit
def kernel(hidden_states, gate_weight):
    b, s, d = hidden_states.shape
    n_tok = b * s
    x2d = hidden_states.reshape(n_tok, d)
    wt = gate_weight.T

    grid = (n_tok // BLK,)
    rw, idx = pl.pallas_call(
        _router_body,
        grid=grid,
        in_specs=[
            pl.BlockSpec((BLK, d), lambda i: (i, 0)),
            pl.BlockSpec((d, EXPERTS), lambda i: (0, 0)),
        ],
        out_specs=[
            pl.BlockSpec((BLK, K), lambda i: (i, 0)),
            pl.BlockSpec((BLK, K), lambda i: (i, 0)),
        ],
        out_shape=[
            jax.ShapeDtypeStruct((n_tok, K), jnp.float32),
            jax.ShapeDtypeStruct((n_tok, K), jnp.int32),
        ],
        compiler_params=pltpu.CompilerParams(
            dimension_semantics=("parallel",),
        ),
    )(x2d, wt)

    return rw.reshape(b, s, K), idx.reshape(b, s, K)


# speedup vs baseline: 2.0360x; 1.0477x over previous
import functools

import jax
import jax.numpy as jnp
from jax.experimental import pallas as pl
from jax.experimental.pallas import tpu as pltpu

HIDDEN = 4096
EXPERTS = 64
K = 8
BLK = 1024
CHUNK = 256


def _router_body(x_ref, w_ref, rw_ref, idx_ref):
    w = w_ref[...]
    riota = (
        jnp.int32(EXPERTS - 1)
        - jax.lax.broadcasted_iota(jnp.int32, (CHUNK, EXPERTS), 1)
    ).astype(jnp.float32)

    for c in range(BLK // CHUNK):
        sl = pl.ds(c * CHUNK, CHUNK)
        logits = jnp.dot(x_ref[sl, :], w, preferred_element_type=jnp.float32)

        vals = logits
        top_vals = []
        top_ridx = []
        for j in range(K):
            m = jnp.max(vals, axis=-1, keepdims=True)
            masked_iota = jnp.where(vals == m, riota, -1.0)
            r = jnp.max(masked_iota, axis=-1, keepdims=True)
            top_vals.append(m)
            top_ridx.append(r)
            if j < K - 1:
                vals = jnp.where(riota == r, -jnp.inf, vals)

        exps = [jnp.ones((CHUNK, 1), jnp.float32)]
        for j in range(1, K):
            exps.append(jnp.exp(top_vals[j] - top_vals[0]))
        denom = exps[0]
        for j in range(1, K):
            denom = denom + exps[j]
        rdenom = 1.0 / denom
        for j in range(K):
            rw_ref[sl, pl.ds(j, 1)] = exps[j] * rdenom
            idx_ref[sl, pl.ds(j, 1)] = (
                jnp.float32(EXPERTS - 1) - top_ridx[j]
            ).astype(jnp.int32)


@jax.jit
def kernel(hidden_states, gate_weight):
    b, s, d = hidden_states.shape
    n_tok = b * s
    x2d = hidden_states.reshape(n_tok, d)
    wt = gate_weight.T

    grid = (n_tok // BLK,)
    rw, idx = pl.pallas_call(
        _router_body,
        grid=grid,
        in_specs=[
            pl.BlockSpec((BLK, d), lambda i: (i, 0)),
            pl.BlockSpec((d, EXPERTS), lambda i: (0, 0)),
        ],
        out_specs=[
            pl.BlockSpec((BLK, K), lambda i: (i, 0)),
            pl.BlockSpec((BLK, K), lambda i: (i, 0)),
        ],
        out_shape=[
            jax.ShapeDtypeStruct((n_tok, K), jnp.float32),
            jax.ShapeDtypeStruct((n_tok, K), jnp.int32),
        ],
        compiler_params=pltpu.CompilerParams(
            dimension_semantics=("parallel",),
        ),
    )(x2d, wt)

    return rw.reshape(b, s, K), idx.reshape(b, s, K)


# speedup vs baseline: 2.0462x; 1.0050x over previous
import functools

import jax
import jax.numpy as jnp
from jax.experimental import pallas as pl
from jax.experimental.pallas import tpu as pltpu

HIDDEN = 4096
EXPERTS = 64
K = 8
BLK = 1024
CHUNK = 128


def _router_body(x_ref, w_ref, rw_ref, idx_ref):
    w = w_ref[...]
    riota = (
        jnp.int32(EXPERTS - 1)
        - jax.lax.broadcasted_iota(jnp.int32, (CHUNK, EXPERTS), 1)
    ).astype(jnp.float32)

    for c in range(BLK // CHUNK):
        sl = pl.ds(c * CHUNK, CHUNK)
        logits = jnp.dot(x_ref[sl, :], w, preferred_element_type=jnp.float32)

        vals = logits
        top_vals = []
        top_ridx = []
        for j in range(K):
            m = jnp.max(vals, axis=-1, keepdims=True)
            masked_iota = jnp.where(vals == m, riota, -1.0)
            r = jnp.max(masked_iota, axis=-1, keepdims=True)
            top_vals.append(m)
            top_ridx.append(r)
            if j < K - 1:
                vals = jnp.where(riota == r, -jnp.inf, vals)

        exps = [jnp.ones((CHUNK, 1), jnp.float32)]
        for j in range(1, K):
            exps.append(jnp.exp(top_vals[j] - top_vals[0]))
        denom = exps[0]
        for j in range(1, K):
            denom = denom + exps[j]
        rdenom = 1.0 / denom
        for j in range(K):
            rw_ref[sl, pl.ds(j, 1)] = exps[j] * rdenom
            idx_ref[sl, pl.ds(j, 1)] = (
                jnp.float32(EXPERTS - 1) - top_ridx[j]
            ).astype(jnp.int32)


@jax.jit
def kernel(hidden_states, gate_weight):
    b, s, d = hidden_states.shape
    n_tok = b * s
    x2d = hidden_states.reshape(n_tok, d)
    wt = gate_weight.T

    grid = (n_tok // BLK,)
    rw, idx = pl.pallas_call(
        _router_body,
        grid=grid,
        in_specs=[
            pl.BlockSpec((BLK, d), lambda i: (i, 0)),
            pl.BlockSpec((d, EXPERTS), lambda i: (0, 0)),
        ],
        out_specs=[
            pl.BlockSpec((BLK, K), lambda i: (i, 0)),
            pl.BlockSpec((BLK, K), lambda i: (i, 0)),
        ],
        out_shape=[
            jax.ShapeDtypeStruct((n_tok, K), jnp.float32),
            jax.ShapeDtypeStruct((n_tok, K), jnp.int32),
        ],
        compiler_params=pltpu.CompilerParams(
            dimension_semantics=("parallel",),
        ),
    )(x2d, wt)

    return rw.reshape(b, s, K), idx.reshape(b, s, K)
